# baseline (speedup 1.0000x reference)
import functools

import jax
import jax.numpy as jnp
from jax import lax
from jax.experimental import pallas as pl
from jax.experimental.pallas import tpu as pltpu
from jax.experimental.pallas import tpu_sc as plsc

BATCH = 16384
BAG_LEN = 50
D = 64
VOCAB = 1000000

NC = 2
NS = 16
NW = NC * NS
BAGS_PER_W = BATCH // NW
STEP_BAGS = 2
ROWS_PER_STEP = STEP_BAGS * BAG_LEN
NSTEP = BAGS_PER_W // STEP_BAGS
NBUF = 8
UNROLL = 5

WPR = D // 2
REPACK_CHUNK = 8192
QUARTER = REPACK_CHUNK // 4
NBLK = (VOCAB + REPACK_CHUNK - 1) // REPACK_CHUNK
VOCAB2 = NBLK * REPACK_CHUNK


def _repack_tc(table_T):
    def body(x_ref, o_ref):
        x = x_ref[...].astype(jnp.bfloat16)
        eye = jnp.eye(D, dtype=jnp.bfloat16)
        xt = lax.dot_general(x, eye, (((0,), (0,)), ((), ())),
                             preferred_element_type=jnp.float32)
        parts = []
        for k in range(4):
            q = xt[k * QUARTER : (k + 1) * QUARTER]
            lo = lax.bitcast_convert_type(q[:, :WPR], jnp.uint32)
            hi = lax.bitcast_convert_type(q[:, WPR:], jnp.uint32)
            word = jnp.bitwise_or(
                lax.shift_right_logical(lo, jnp.uint32(16)),
                jnp.bitwise_and(hi, jnp.uint32(0xFFFF0000)),
            )
            parts.append(lax.bitcast_convert_type(word, jnp.int32))
        o_ref[...] = jnp.concatenate(parts, axis=1)

    return pl.pallas_call(
        body,
        out_shape=jax.ShapeDtypeStruct((NBLK * QUARTER, 2 * D), jnp.int32),
        grid=(NBLK,),
        in_specs=[pl.BlockSpec((D, REPACK_CHUNK), lambda i: (0, i))],
        out_specs=pl.BlockSpec((QUARTER, 2 * D), lambda i: (i, 0)),
        compiler_params=pltpu.CompilerParams(fuse_transposed_lhs_in_matmul=True),
    )(table_T)


def _bag_sum_sc(tokens2d, table_pk):
    mesh = plsc.VectorSubcoreMesh(core_axis_name="c", subcore_axis_name="s")

    @functools.partial(
        pl.kernel,
        out_type=jax.ShapeDtypeStruct((D, BATCH // 128, 128), jnp.float32),
        mesh=mesh,
        scratch_types=[
            pltpu.VMEM((NSTEP, ROWS_PER_STEP), jnp.int32),
            pltpu.VMEM((NBUF, ROWS_PER_STEP, WPR), jnp.int32),
            pltpu.VMEM((D, BAGS_PER_W // 128, 128), jnp.float32),
            pltpu.SemaphoreType.DMA,
        ],
        compiler_params=pltpu.CompilerParams(
            use_tc_tiling_on_sc=False, needs_layout_passes=False
        ),
    )
    def kern(tokens_hbm, table_hbm, out_hbm, idx_v, ring_v, pooled_v, sem):
        wid = lax.axis_index("s") * NC + lax.axis_index("c")
        row_base = wid * NSTEP

        pltpu.sync_copy(tokens_hbm.at[pl.ds(row_base, NSTEP)], idx_v)

        for s in range(NBUF):
            pltpu.async_copy(table_hbm.at[idx_v.at[s]], ring_v.at[s], sem)

        himask = jnp.full((16,), -65536, jnp.int32)
        dim_idx = [jnp.arange(16, dtype=jnp.int32) + 16 * q for q in range(4)]

        def accumulate(slot, bag, j):
            def body(i, carry):
                a0, a1, a2, a3 = carry
                for u in range(UNROLL):
                    r = bag * BAG_LEN + i * UNROLL + u
                    w0 = ring_v[slot, r, pl.ds(0, 16)]
                    w1 = ring_v[slot, r, pl.ds(16, 16)]
                    a0 = a0 + plsc.bitcast(w0 << 16, jnp.float32)
                    a2 = a2 + plsc.bitcast(w0 & himask, jnp.float32)
                    a1 = a1 + plsc.bitcast(w1 << 16, jnp.float32)
                    a3 = a3 + plsc.bitcast(w1 & himask, jnp.float32)
                return (a0, a1, a2, a3)

            zeros = tuple(jnp.zeros((16,), jnp.float32) for _ in range(4))
            accs = lax.fori_loop(0, BAG_LEN // UNROLL, body, zeros)
            bag_local = j * STEP_BAGS + bag
            c_idx = jnp.full((16,), bag_local // 128, jnp.int32)
            j_idx = jnp.full((16,), bag_local % 128, jnp.int32)
            for q in range(4):
                plsc.store_scatter(pooled_v, [dim_idx[q], c_idx, j_idx], accs[q])

        @pl.loop(0, NSTEP, step=NBUF)
        def _steps(j0):
            for s in range(NBUF):
                j = j0 + s
                pltpu.make_async_copy(
                    table_hbm.at[pl.ds(0, ROWS_PER_STEP)], ring_v.at[s], sem
                ).wait()
                for bag in range(STEP_BAGS):
                    accumulate(s, bag, j)
                nj = j + NBUF

                @pl.when(nj < NSTEP)
                def _():
                    pltpu.async_copy(
                        table_hbm.at[idx_v.at[nj]], ring_v.at[s], sem
                    )

        pltpu.sync_copy(
            pooled_v, out_hbm.at[:, pl.ds(wid * (BAGS_PER_W // 128), BAGS_PER_W // 128), :]
        )

    return kern(tokens2d, table_pk)


def _project_tc(pooled_T3, W, b2d):
    NCHUNK = BATCH // 128
    CPB = 8

    def body(p_ref, w_ref, b_ref, o_ref):
        ws = w_ref[...] * (1.0 / BAG_LEN)
        y3 = lax.dot_general(
            ws, p_ref[...], (((1,), (0,)), ((), ())),
            preferred_element_type=jnp.float32,
        ) + b_ref[...][:, :, None]
        for k in range(CPB):
            o_ref[:, k, :, :] = y3[:, k, :].reshape(D // 8, 8, 128)

    return pl.pallas_call(
        body,
        out_shape=jax.ShapeDtypeStruct((D // 8, NCHUNK, 8, 128), jnp.float32),
        grid=(NCHUNK // CPB,),
        in_specs=[
            pl.BlockSpec((D, CPB, 128), lambda i: (0, i, 0)),
            pl.BlockSpec((D, D), lambda i: (0, 0)),
            pl.BlockSpec((D, 1), lambda i: (0, 0)),
        ],
        out_specs=pl.BlockSpec((D // 8, CPB, 8, 128), lambda i: (0, i, 0, 0)),
    )(pooled_T3, W, b2d)


@jax.jit
def kernel(tokens, table, W, b):
    packed = _repack_tc(table.T)
    packed = jax.lax.optimization_barrier(packed)
    table_pk = packed.reshape(VOCAB2, WPR)
    t = tokens.astype(jnp.int32)
    C = REPACK_CHUNK
    rows = 4 * (QUARTER * (t // C) + t % QUARTER) + (t % C) // QUARTER
    tokens2d = rows.reshape(NW * NSTEP, ROWS_PER_STEP)
    pooled_T3 = _bag_sum_sc(tokens2d, table_pk)
    out4 = _project_tc(pooled_T3, W, b.reshape(D, 1))
    return out4.transpose(1, 3, 0, 2).reshape(BATCH, D)


# speedup vs baseline: 1.0234x; 1.0234x over previous
import functools

import jax
import jax.numpy as jnp
from jax import lax
from jax.experimental import pallas as pl
from jax.experimental.pallas import tpu as pltpu
from jax.experimental.pallas import tpu_sc as plsc

BATCH = 16384
BAG_LEN = 50
D = 64
VOCAB = 1000000

NC = 2
NS = 16
NW = NC * NS
BAGS_PER_W = BATCH // NW
STEP_BAGS = 2
ROWS_PER_STEP = STEP_BAGS * BAG_LEN
NSTEP = BAGS_PER_W // STEP_BAGS
NBUF = 8
UNROLL = 5

WPR = D // 2
REPACK_CHUNK = 8192
QUARTER = REPACK_CHUNK // 4
NBLK = (VOCAB + REPACK_CHUNK - 1) // REPACK_CHUNK
VOCAB2 = NBLK * REPACK_CHUNK


def _repack_tc(table_T):
    def body(x_ref, o_ref):
        x = x_ref[...].astype(jnp.bfloat16)
        eye = jnp.eye(D, dtype=jnp.bfloat16)
        xt = lax.dot_general(x, eye, (((0,), (0,)), ((), ())),
                             preferred_element_type=jnp.float32)
        parts = []
        for k in range(4):
            q = xt[k * QUARTER : (k + 1) * QUARTER]
            lo = lax.bitcast_convert_type(q[:, :WPR], jnp.uint32)
            hi = lax.bitcast_convert_type(q[:, WPR:], jnp.uint32)
            word = jnp.bitwise_or(
                lax.shift_right_logical(lo, jnp.uint32(16)),
                jnp.bitwise_and(hi, jnp.uint32(0xFFFF0000)),
            )
            parts.append(lax.bitcast_convert_type(word, jnp.int32))
        o_ref[...] = jnp.concatenate(parts, axis=1)

    return pl.pallas_call(
        body,
        out_shape=jax.ShapeDtypeStruct((NBLK * QUARTER, 2 * D), jnp.int32),
        grid=(NBLK,),
        in_specs=[pl.BlockSpec((D, REPACK_CHUNK), lambda i: (0, i))],
        out_specs=pl.BlockSpec((QUARTER, 2 * D), lambda i: (i, 0)),
        compiler_params=pltpu.CompilerParams(fuse_transposed_lhs_in_matmul=True),
    )(table_T)


def _bag_sum_sc(tokens2d, table_pk):
    mesh = plsc.VectorSubcoreMesh(core_axis_name="c", subcore_axis_name="s")

    @functools.partial(
        pl.kernel,
        out_type=jax.ShapeDtypeStruct((BATCH, D), jnp.float32),
        mesh=mesh,
        scratch_types=[
            pltpu.VMEM((NSTEP, ROWS_PER_STEP), jnp.int32),
            pltpu.VMEM((NBUF, ROWS_PER_STEP, WPR), jnp.int32),
            pltpu.VMEM((BAGS_PER_W, D), jnp.float32),
            pltpu.SemaphoreType.DMA,
        ],
        compiler_params=pltpu.CompilerParams(
            use_tc_tiling_on_sc=False, needs_layout_passes=False
        ),
    )
    def kern(tokens_hbm, table_hbm, out_hbm, idx_v, ring_v, pooled_v, sem):
        wid = lax.axis_index("s") * NC + lax.axis_index("c")
        row_base = wid * NSTEP

        pltpu.sync_copy(tokens_hbm.at[pl.ds(row_base, NSTEP)], idx_v)

        for s in range(NBUF):
            pltpu.async_copy(table_hbm.at[idx_v.at[s]], ring_v.at[s], sem)

        himask = jnp.full((16,), -65536, jnp.int32)

        def accumulate(slot, bag, j):
            def body(i, carry):
                a0, a1, a2, a3 = carry
                for u in range(UNROLL):
                    r = bag * BAG_LEN + i * UNROLL + u
                    w0 = ring_v[slot, r, pl.ds(0, 16)]
                    w1 = ring_v[slot, r, pl.ds(16, 16)]
                    a0 = a0 + plsc.bitcast(w0 << 16, jnp.float32)
                    a2 = a2 + plsc.bitcast(w0 & himask, jnp.float32)
                    a1 = a1 + plsc.bitcast(w1 << 16, jnp.float32)
                    a3 = a3 + plsc.bitcast(w1 & himask, jnp.float32)
                return (a0, a1, a2, a3)

            zeros = tuple(jnp.zeros((16,), jnp.float32) for _ in range(4))
            accs = lax.fori_loop(0, BAG_LEN // UNROLL, body, zeros)
            for q in range(4):
                pooled_v[j * STEP_BAGS + bag, pl.ds(q * 16, 16)] = accs[q]

        @pl.loop(0, NSTEP, step=NBUF)
        def _steps(j0):
            for s in range(NBUF):
                j = j0 + s
                pltpu.make_async_copy(
                    table_hbm.at[pl.ds(0, ROWS_PER_STEP)], ring_v.at[s], sem
                ).wait()
                for bag in range(STEP_BAGS):
                    accumulate(s, bag, j)
                nj = j + NBUF

                @pl.when(nj < NSTEP)
                def _():
                    pltpu.async_copy(
                        table_hbm.at[idx_v.at[nj]], ring_v.at[s], sem
                    )

        pltpu.sync_copy(pooled_v, out_hbm.at[pl.ds(wid * BAGS_PER_W, BAGS_PER_W)])

    return kern(tokens2d, table_pk)


def _project_tc(pooled_sum, W, b2d):
    BLK = 2048

    def body(p_ref, w_ref, b_ref, o_ref):
        x = p_ref[...] * (1.0 / BAG_LEN)
        o_ref[...] = (
            lax.dot_general(
                x, w_ref[...], (((1,), (1,)), ((), ())),
                preferred_element_type=jnp.float32,
            )
            + b_ref[...]
        )

    return pl.pallas_call(
        body,
        out_shape=jax.ShapeDtypeStruct((BATCH, D), jnp.float32),
        grid=(BATCH // BLK,),
        in_specs=[
            pl.BlockSpec((BLK, D), lambda i: (i, 0)),
            pl.BlockSpec((D, D), lambda i: (0, 0)),
            pl.BlockSpec((1, D), lambda i: (0, 0)),
        ],
        out_specs=pl.BlockSpec((BLK, D), lambda i: (i, 0)),
    )(pooled_sum, W, b2d)


@jax.jit
def kernel(tokens, table, W, b):
    packed = _repack_tc(table.T)
    packed = jax.lax.optimization_barrier(packed)
    table_pk = packed.reshape(VOCAB2, WPR)
    t = tokens.astype(jnp.int32)
    C = REPACK_CHUNK
    rows = 4 * (QUARTER * (t // C) + t % QUARTER) + (t % C) // QUARTER
    tokens2d = rows.reshape(NW * NSTEP, ROWS_PER_STEP)
    pooled_sum = _bag_sum_sc(tokens2d, table_pk)
    return _project_tc(pooled_sum, W, b.reshape(1, D))


# speedup vs baseline: 1.0562x; 1.0321x over previous
import functools

import jax
import jax.numpy as jnp
from jax import lax
from jax.experimental import pallas as pl
from jax.experimental.pallas import tpu as pltpu
from jax.experimental.pallas import tpu_sc as plsc

BATCH = 16384
BAG_LEN = 50
D = 64
VOCAB = 1000000

NC = 2
NS = 16
NW = NC * NS
BAGS_PER_W = BATCH // NW
STEP_BAGS = 2
ROWS_PER_STEP = STEP_BAGS * BAG_LEN
NSTEP = BAGS_PER_W // STEP_BAGS
NBUF = 8
UNROLL = 5

WPR = D // 2
REPACK_CHUNK = 16384
QUARTER = REPACK_CHUNK // 4
NBLK = (VOCAB + REPACK_CHUNK - 1) // REPACK_CHUNK
VOCAB2 = NBLK * REPACK_CHUNK


def _repack_tc(table_T):
    def body(x_ref, o_ref):
        x = x_ref[...].astype(jnp.bfloat16)
        eye = jnp.eye(D, dtype=jnp.bfloat16)
        xt = lax.dot_general(x, eye, (((0,), (0,)), ((), ())),
                             preferred_element_type=jnp.float32)
        parts = []
        for k in range(4):
            q = xt[k * QUARTER : (k + 1) * QUARTER]
            lo = lax.bitcast_convert_type(q[:, :WPR], jnp.uint32)
            hi = lax.bitcast_convert_type(q[:, WPR:], jnp.uint32)
            word = jnp.bitwise_or(
                lax.shift_right_logical(lo, jnp.uint32(16)),
                jnp.bitwise_and(hi, jnp.uint32(0xFFFF0000)),
            )
            parts.append(lax.bitcast_convert_type(word, jnp.int32))
        o_ref[...] = jnp.concatenate(parts, axis=1)

    return pl.pallas_call(
        body,
        out_shape=jax.ShapeDtypeStruct((NBLK * QUARTER, 2 * D), jnp.int32),
        grid=(NBLK,),
        in_specs=[pl.BlockSpec((D, REPACK_CHUNK), lambda i: (0, i))],
        out_specs=pl.BlockSpec((QUARTER, 2 * D), lambda i: (i, 0)),
        compiler_params=pltpu.CompilerParams(fuse_transposed_lhs_in_matmul=True),
    )(table_T)


def _bag_sum_sc(tokens2d, table_pk):
    mesh = plsc.VectorSubcoreMesh(core_axis_name="c", subcore_axis_name="s")

    @functools.partial(
        pl.kernel,
        out_type=jax.ShapeDtypeStruct((BATCH, D), jnp.float32),
        mesh=mesh,
        scratch_types=[
            pltpu.VMEM((NSTEP, ROWS_PER_STEP), jnp.int32),
            pltpu.VMEM((NBUF, ROWS_PER_STEP, WPR), jnp.int32),
            pltpu.VMEM((BAGS_PER_W, D), jnp.float32),
            pltpu.SemaphoreType.DMA,
        ],
        compiler_params=pltpu.CompilerParams(
            use_tc_tiling_on_sc=False, needs_layout_passes=False
        ),
    )
    def kern(tokens_hbm, table_hbm, out_hbm, idx_v, ring_v, pooled_v, sem):
        wid = lax.axis_index("s") * NC + lax.axis_index("c")
        row_base = wid * NSTEP

        pltpu.sync_copy(tokens_hbm.at[pl.ds(row_base, NSTEP)], idx_v)

        for s in range(NBUF):
            pltpu.async_copy(table_hbm.at[idx_v.at[s]], ring_v.at[s], sem)

        himask = jnp.full((16,), -65536, jnp.int32)

        def accumulate(slot, bag, j):
            def body(i, carry):
                a0, a1, a2, a3 = carry
                for u in range(UNROLL):
                    r = bag * BAG_LEN + i * UNROLL + u
                    w0 = ring_v[slot, r, pl.ds(0, 16)]
                    w1 = ring_v[slot, r, pl.ds(16, 16)]
                    a0 = a0 + plsc.bitcast(w0 << 16, jnp.float32)
                    a2 = a2 + plsc.bitcast(w0 & himask, jnp.float32)
                    a1 = a1 + plsc.bitcast(w1 << 16, jnp.float32)
                    a3 = a3 + plsc.bitcast(w1 & himask, jnp.float32)
                return (a0, a1, a2, a3)

            zeros = tuple(jnp.zeros((16,), jnp.float32) for _ in range(4))
            accs = lax.fori_loop(0, BAG_LEN // UNROLL, body, zeros)
            for q in range(4):
                pooled_v[j * STEP_BAGS + bag, pl.ds(q * 16, 16)] = accs[q]

        @pl.loop(0, NSTEP, step=NBUF)
        def _steps(j0):
            for s in range(NBUF):
                j = j0 + s
                pltpu.make_async_copy(
                    table_hbm.at[pl.ds(0, ROWS_PER_STEP)], ring_v.at[s], sem
                ).wait()
                for bag in range(STEP_BAGS):
                    accumulate(s, bag, j)
                nj = j + NBUF

                @pl.when(nj < NSTEP)
                def _():
                    pltpu.async_copy(
                        table_hbm.at[idx_v.at[nj]], ring_v.at[s], sem
                    )

        pltpu.sync_copy(pooled_v, out_hbm.at[pl.ds(wid * BAGS_PER_W, BAGS_PER_W)])

    return kern(tokens2d, table_pk)


def _project_tc(pooled_sum, W, b2d):
    BLK = 2048

    def body(p_ref, w_ref, b_ref, o_ref):
        x = p_ref[...] * (1.0 / BAG_LEN)
        o_ref[...] = (
            lax.dot_general(
                x, w_ref[...], (((1,), (1,)), ((), ())),
                preferred_element_type=jnp.float32,
            )
            + b_ref[...]
        )

    return pl.pallas_call(
        body,
        out_shape=jax.ShapeDtypeStruct((BATCH, D), jnp.float32),
        grid=(BATCH // BLK,),
        in_specs=[
            pl.BlockSpec((BLK, D), lambda i: (i, 0)),
            pl.BlockSpec((D, D), lambda i: (0, 0)),
            pl.BlockSpec((1, D), lambda i: (0, 0)),
        ],
        out_specs=pl.BlockSpec((BLK, D), lambda i: (i, 0)),
    )(pooled_sum, W, b2d)


@jax.jit
def kernel(tokens, table, W, b):
    packed = _repack_tc(table.T)
    packed = jax.lax.optimization_barrier(packed)
    table_pk = packed.reshape(VOCAB2, WPR)
    t = tokens.astype(jnp.int32)
    C = REPACK_CHUNK
    rows = 4 * (QUARTER * (t // C) + t % QUARTER) + (t % C) // QUARTER
    tokens2d = rows.reshape(NW * NSTEP, ROWS_PER_STEP)
    pooled_sum = _bag_sum_sc(tokens2d, table_pk)
    return _project_tc(pooled_sum, W, b.reshape(1, D))
